# Initial kernel scaffold; baseline (speedup 1.0000x reference)
import jax
import jax.numpy as jnp
from jax.experimental import pallas as pl


def kernel(hidden_states, v_first, x_r, x_w, x_k, x_v, x_a, x_g, k_k, k_a, r_k, W_r, W_k, W_v, W_o, w_lora_a, w_lora_b, w_lora_bias, a_lora_a, a_lora_b, a_lora_bias, v_lora_a, v_lora_b, v_lora_bias, g_lora_a, g_lora_b, gn_gamma, gn_beta):
    raise NotImplementedError("write your pallas kernel here")


# speedup vs baseline: 4.2468x; 4.2468x over previous
import functools

import jax
import jax.numpy as jnp
from jax.experimental import pallas as pl
from jax.experimental.pallas import tpu as pltpu

D = 1024
HEADS = 16
HK = 64
HV = 64
GN_EPS = HK * 1e-5
W_SCALE = -0.6065306597126334

BS = 256
C = 64

_HI = jax.lax.Precision.HIGHEST


def _dot_t(x, w, precision=None):
    return jax.lax.dot_general(x, w, (((1,), (1,)), ((), ())),
                               precision=precision,
                               preferred_element_type=jnp.float32)


def _dot(x, w, precision=None):
    return jax.lax.dot_general(x, w, (((1,), (0,)), ((), ())),
                               precision=precision,
                               preferred_element_type=jnp.float32)


def _proj_kernel(x_ref, xs_ref, vf_ref,
                 mixes_ref, kka_ref,
                 wr_ref, wk_ref, wv_ref,
                 wla_ref, wlb_ref, wlbias_ref,
                 ala_ref, alb_ref, albias_ref,
                 vla_ref, vlb_ref, vlbias_ref,
                 gla_ref, glb_ref,
                 r_out, w_out, k_out, v_out, am_out, bm_out, g_out):
    x = x_ref[0]
    delta = xs_ref[0] - x
    mixes = mixes_ref[...]
    xr = x + delta * mixes[0:1, :]
    xw = x + delta * mixes[1:2, :]
    xk = x + delta * mixes[2:3, :]
    xv = x + delta * mixes[3:4, :]
    xa = x + delta * mixes[4:5, :]
    xg = x + delta * mixes[5:6, :]

    r = _dot_t(xr, wr_ref[...])
    k0 = _dot_t(xk, wk_ref[...])
    v0 = _dot_t(xv, wv_ref[...])

    w_log = W_SCALE * jax.nn.sigmoid(
        _dot_t(jnp.tanh(_dot_t(xw, wla_ref[...], _HI)), wlb_ref[...], _HI)
        + wlbias_ref[...])
    a_sig = jax.nn.sigmoid(
        _dot_t(_dot_t(xa, ala_ref[...], _HI), alb_ref[...], _HI)
        + albias_ref[...])
    v_mix = jax.nn.sigmoid(
        _dot_t(_dot_t(xv, vla_ref[...], _HI), vlb_ref[...], _HI)
        + vlbias_ref[...])
    g = _dot_t(jax.nn.sigmoid(_dot_t(xg, gla_ref[...], _HI)), glb_ref[...], _HI)

    v = v0 + v_mix * (vf_ref[0] - v0)

    k_k = kka_ref[0:1, :]
    k_a = kka_ref[1:2, :]
    kk = k0 * k_k
    ch = jax.lax.broadcasted_iota(jnp.int32, (D, HEADS), 0) // HK
    hh = jax.lax.broadcasted_iota(jnp.int32, (D, HEADS), 1)
    mask_dh = jnp.where(ch == hh, 1.0, 0.0)
    s2h = _dot(kk * kk, mask_dh, _HI)
    s2 = _dot_t(s2h, mask_dh, _HI)
    kk_n = kk / jnp.maximum(jnp.sqrt(s2), 1e-12)

    k = k0 * (1.0 + (a_sig - 1.0) * k_a)

    r_out[0] = r
    w_out[0] = w_log
    k_out[0] = k
    v_out[0] = v
    am_out[0] = -kk_n
    bm_out[0] = kk_n * a_sig
    g_out[0] = g


def _scan_kernel(r_ref, w_ref, k_ref, v_ref, am_ref, bm_ref, g_ref,
                 wo_ref, rk_ref, gn_ref,
                 out_ref, s_sc, y_sc):
    ci = pl.program_id(1)

    @pl.when(ci == 0)
    def _():
        s_sc[...] = jnp.zeros_like(s_sc)

    i0 = jax.lax.broadcasted_iota(jnp.int32, (C, C), 0)
    i1 = jax.lax.broadcasted_iota(jnp.int32, (C, C), 1)
    incl = i0 >= i1
    strict = i0 > i1
    l_incl = jnp.where(incl, 1.0, 0.0)
    eye = jnp.where(i0 == i1, 1.0, 0.0)

    r_all = r_ref[0]
    w_all = w_ref[0]
    k_all = k_ref[0]
    v_all = v_ref[0]
    a_all = am_ref[0]
    b_all = bm_ref[0]

    for h in range(HEADS):
        sl = slice(h * HK, (h + 1) * HK)
        rc = r_all[:, sl]
        wc = w_all[:, sl]
        kc = k_all[:, sl]
        vc = v_all[:, sl]
        ac = a_all[:, sl]
        bc = b_all[:, sl]
        S = s_sc[sl, :]

        c = _dot(l_incl, wc, _HI)
        ec = jnp.exp(c)
        ec_inv = 1.0 / ec
        rhat = rc * ec
        ktil = kc * ec_inv
        btil = bc * ec_inv
        ahat = ac * jnp.exp(c - wc)

        m_ab = jnp.where(strict, _dot_t(ahat, btil, _HI), 0.0)
        m_ak = jnp.where(strict, _dot_t(ahat, ktil, _HI), 0.0)

        tm = eye + m_ab
        mp = m_ab
        for _ in range(5):
            mp = _dot(mp, mp, _HI)
            tm = _dot(eye + mp, tm, _HI)

        rhs = _dot(m_ak, vc, _HI) + _dot(ahat, S, _HI)
        u = _dot(tm, rhs, _HI)

        o_rb = jnp.where(incl, _dot_t(rhat, btil, _HI), 0.0)
        o_rk = jnp.where(incl, _dot_t(rhat, ktil, _HI), 0.0)
        o = _dot(o_rb, u, _HI) + _dot(o_rk, vc, _HI) + _dot(rhat, S, _HI)

        clast = c[C - 1:C, :]
        dl = jnp.exp(clast - c)
        s_new = (S * jnp.transpose(jnp.exp(clast))
                 + jax.lax.dot_general(bc * dl, u, (((0,), (0,)), ((), ())),
                                       precision=_HI,
                                       preferred_element_type=jnp.float32)
                 + jax.lax.dot_general(kc * dl, vc, (((0,), (0,)), ((), ())),
                                       precision=_HI,
                                       preferred_element_type=jnp.float32))
        s_sc[sl, :] = s_new

        mean = jnp.mean(o, axis=-1, keepdims=True)
        xc = o - mean
        var = jnp.mean(xc * xc, axis=-1, keepdims=True)
        o_n = xc * jax.lax.rsqrt(var + GN_EPS)
        gamma = gn_ref[0:1, sl]
        beta = gn_ref[1:2, sl]
        rk_row = rk_ref[h:h + 1, :]
        bonus = jnp.sum(rc * kc * rk_row, axis=-1, keepdims=True) * vc
        y_sc[:, sl] = o_n * gamma + beta + bonus

    yg = y_sc[...] * g_ref[0]
    out_ref[0] = _dot_t(yg, wo_ref[...])


@jax.jit
def kernel(hidden_states, v_first, x_r, x_w, x_k, x_v, x_a, x_g, k_k, k_a,
           r_k, W_r, W_k, W_v, W_o, w_lora_a, w_lora_b, w_lora_bias,
           a_lora_a, a_lora_b, a_lora_bias, v_lora_a, v_lora_b, v_lora_bias,
           g_lora_a, g_lora_b, gn_gamma, gn_beta):
    B, T, Dm = hidden_states.shape
    x = hidden_states
    shifted = jnp.pad(x, ((0, 0), (1, 0), (0, 0)))[:, :T]
    mixes = jnp.stack([x_r, x_w, x_k, x_v, x_a, x_g], axis=0)
    kka = jnp.stack([k_k, k_a], axis=0)
    biases = [w_lora_bias.reshape(1, Dm), a_lora_bias.reshape(1, Dm),
              v_lora_bias.reshape(1, Dm)]
    gn = jnp.stack([gn_gamma, gn_beta], axis=0)

    nbt = T // BS
    row_spec = pl.BlockSpec((1, BS, Dm), lambda b, t: (b, t, 0))
    full = lambda s: pl.BlockSpec(s, lambda b, t: tuple(0 for _ in s))
    outs = pl.pallas_call(
        _proj_kernel,
        grid=(B, nbt),
        in_specs=[row_spec, row_spec, row_spec,
                  full((6, Dm)), full((2, Dm)),
                  full(W_r.shape), full(W_k.shape), full(W_v.shape),
                  full(w_lora_a.shape), full(w_lora_b.shape), full((1, Dm)),
                  full(a_lora_a.shape), full(a_lora_b.shape), full((1, Dm)),
                  full(v_lora_a.shape), full(v_lora_b.shape), full((1, Dm)),
                  full(g_lora_a.shape), full(g_lora_b.shape)],
        out_specs=[row_spec] * 7,
        out_shape=[jax.ShapeDtypeStruct((B, T, Dm), jnp.float32)] * 7,
        compiler_params=pltpu.CompilerParams(
            dimension_semantics=("parallel", "parallel"),
            vmem_limit_bytes=60 * 1024 * 1024,
        ),
    )(x, shifted, v_first, mixes, kka, W_r, W_k, W_v,
      w_lora_a, w_lora_b, biases[0], a_lora_a, a_lora_b, biases[1],
      v_lora_a, v_lora_b, biases[2], g_lora_a, g_lora_b)
    r, w, k, v, am, bm, g = outs

    nc = T // C
    chunk_spec = pl.BlockSpec((1, C, Dm), lambda b, t: (b, t, 0))
    out = pl.pallas_call(
        _scan_kernel,
        grid=(B, nc),
        in_specs=[chunk_spec] * 6 + [chunk_spec,
                  full(W_o.shape), full(r_k.shape), full((2, Dm))],
        out_specs=chunk_spec,
        out_shape=jax.ShapeDtypeStruct((B, T, Dm), jnp.float32),
        scratch_shapes=[pltpu.VMEM((HEADS * HK, HV), jnp.float32),
                        pltpu.VMEM((C, Dm), jnp.float32)],
        compiler_params=pltpu.CompilerParams(
            dimension_semantics=("parallel", "arbitrary"),
            vmem_limit_bytes=60 * 1024 * 1024,
        ),
    )(r, w, k, v, am, bm, g, W_o, r_k, gn)
    return out


# speedup vs baseline: 6.4091x; 1.5092x over previous
import functools

import jax
import jax.numpy as jnp
from jax.experimental import pallas as pl
from jax.experimental.pallas import tpu as pltpu

D = 1024
HEADS = 16
HK = 64
HV = 64
GN_EPS = HK * 1e-5
W_SCALE = -0.6065306597126334

BS = 256
C = 64

_HI = jax.lax.Precision.HIGHEST
_BF = jnp.bfloat16


def _dot_t(x, w, precision=None):
    return jax.lax.dot_general(x, w, (((1,), (1,)), ((), ())),
                               precision=precision,
                               preferred_element_type=jnp.float32)


def _dot(x, w, precision=None):
    return jax.lax.dot_general(x, w, (((1,), (0,)), ((), ())),
                               precision=precision,
                               preferred_element_type=jnp.float32)


def _bdot_t(x, y):
    return jax.lax.dot_general(x.astype(_BF), y.astype(_BF),
                               (((1,), (1,)), ((), ())),
                               preferred_element_type=jnp.float32)


def _bdot(x, y):
    return jax.lax.dot_general(x.astype(_BF), y.astype(_BF),
                               (((1,), (0,)), ((), ())),
                               preferred_element_type=jnp.float32)


def _bdot0(x, y):
    return jax.lax.dot_general(x.astype(_BF), y.astype(_BF),
                               (((0,), (0,)), ((), ())),
                               preferred_element_type=jnp.float32)


def _proj_kernel(x_ref, xs_ref, vf_ref,
                 mixes_ref, kka_ref,
                 wr_ref, wk_ref, wv_ref,
                 wla_ref, wlb_ref, wlbias_ref,
                 ala_ref, alb_ref, albias_ref,
                 vla_ref, vlb_ref, vlbias_ref,
                 gla_ref, glb_ref,
                 r_out, w_out, k_out, v_out, am_out, bm_out, g_out):
    x = x_ref[0]
    delta = xs_ref[0] - x
    mixes = mixes_ref[...]
    xr = x + delta * mixes[0:1, :]
    xw = x + delta * mixes[1:2, :]
    xk = x + delta * mixes[2:3, :]
    xv = x + delta * mixes[3:4, :]
    xa = x + delta * mixes[4:5, :]
    xg = x + delta * mixes[5:6, :]

    r = _bdot_t(xr, wr_ref[...])
    k0 = _bdot_t(xk, wk_ref[...])
    v0 = _bdot_t(xv, wv_ref[...])

    w_log = W_SCALE * jax.nn.sigmoid(
        _dot_t(jnp.tanh(_dot_t(xw, wla_ref[...], _HI)), wlb_ref[...], _HI)
        + wlbias_ref[...])
    a_sig = jax.nn.sigmoid(
        _dot_t(_dot_t(xa, ala_ref[...], _HI), alb_ref[...], _HI)
        + albias_ref[...])
    v_mix = jax.nn.sigmoid(
        _dot_t(_dot_t(xv, vla_ref[...], _HI), vlb_ref[...], _HI)
        + vlbias_ref[...])
    g = _dot_t(jax.nn.sigmoid(_dot_t(xg, gla_ref[...], _HI)), glb_ref[...], _HI)

    v = v0 + v_mix * (vf_ref[0] - v0)

    k_k = kka_ref[0:1, :]
    k_a = kka_ref[1:2, :]
    kk = k0 * k_k
    ch = jax.lax.broadcasted_iota(jnp.int32, (D, HEADS), 0) // HK
    hh = jax.lax.broadcasted_iota(jnp.int32, (D, HEADS), 1)
    mask_dh = jnp.where(ch == hh, 1.0, 0.0)
    s2h = _dot(kk * kk, mask_dh, _HI)
    s2 = _dot_t(s2h, mask_dh, _HI)
    kk_n = kk / jnp.maximum(jnp.sqrt(s2), 1e-12)

    k = k0 * (1.0 + (a_sig - 1.0) * k_a)

    r_out[0] = r
    w_out[0] = w_log
    k_out[0] = k
    v_out[0] = v
    am_out[0] = -kk_n
    bm_out[0] = kk_n * a_sig
    g_out[0] = g


def _scan_kernel(r_ref, w_ref, k_ref, v_ref, am_ref, bm_ref, g_ref,
                 wo_ref, rk_ref, gn_ref,
                 out_ref, s_sc, y_sc):
    ci = pl.program_id(1)

    @pl.when(ci == 0)
    def _():
        s_sc[...] = jnp.zeros_like(s_sc)

    i0 = jax.lax.broadcasted_iota(jnp.int32, (C, C), 0)
    i1 = jax.lax.broadcasted_iota(jnp.int32, (C, C), 1)
    l_incl = jnp.where(i0 >= i1, 1.0, 0.0)
    eye = jnp.where(i0 == i1, 1.0, 0.0)
    i02 = jax.lax.broadcasted_iota(jnp.int32, (C, 2 * C), 0)
    j2 = jax.lax.broadcasted_iota(jnp.int32, (C, 2 * C), 1) & (C - 1)
    strict2 = i02 > j2
    incl2 = i02 >= j2

    r_all = r_ref[0]
    w_all = w_ref[0]
    k_all = k_ref[0]
    v_all = v_ref[0]
    a_all = am_ref[0]
    b_all = bm_ref[0]

    for h in range(HEADS):
        sl = slice(h * HK, (h + 1) * HK)
        rc = r_all[:, sl]
        wc = w_all[:, sl]
        kc = k_all[:, sl]
        vc = v_all[:, sl]
        ac = a_all[:, sl]
        bc = b_all[:, sl]
        S = s_sc[sl, :]

        wh = wc.astype(_BF).astype(jnp.float32)
        wl = wc - wh
        c = _bdot(l_incl, wh) + _bdot(l_incl, wl)
        ec = jnp.exp(c)
        ec_inv = 1.0 / ec
        rhat = rc * ec
        ahat = ac * jnp.exp(c - wc)
        bkt = jnp.concatenate([bc * ec_inv, kc * ec_inv], axis=0)

        mm = jnp.where(strict2, _bdot_t(ahat, bkt), 0.0)
        m_ab = mm[:, :C]
        m_ak = mm[:, C:]

        tm = eye + m_ab
        mp = m_ab
        for _ in range(5):
            mp = _bdot(mp, mp)
            tm = _bdot(eye + mp, tm)

        vs = jnp.concatenate([vc, S], axis=0)
        rhs = _bdot(jnp.concatenate([m_ak, ahat], axis=1), vs)
        u = _bdot(tm, rhs)

        rr = jnp.where(incl2, _bdot_t(rhat, bkt), 0.0)
        uvs = jnp.concatenate([u, vs], axis=0)
        o = _bdot(jnp.concatenate([rr, rhat], axis=1), uvs)

        clast = c[C - 1:C, :]
        dl = jnp.exp(clast - c)
        pq = jnp.concatenate([bc * dl, kc * dl], axis=0)
        s_new = (S * jnp.transpose(jnp.exp(clast))
                 + _bdot0(pq, uvs[:2 * C]))
        s_sc[sl, :] = s_new

        mean = jnp.mean(o, axis=-1, keepdims=True)
        xc = o - mean
        var = jnp.mean(xc * xc, axis=-1, keepdims=True)
        o_n = xc * jax.lax.rsqrt(var + GN_EPS)
        gamma = gn_ref[0:1, sl]
        beta = gn_ref[1:2, sl]
        rk_row = rk_ref[h:h + 1, :]
        bonus = jnp.sum(rc * kc * rk_row, axis=-1, keepdims=True) * vc
        y_sc[:, sl] = o_n * gamma + beta + bonus

    yg = y_sc[...] * g_ref[0]
    out_ref[0] = jax.lax.dot_general(yg.astype(_BF), wo_ref[...],
                                     (((1,), (1,)), ((), ())),
                                     preferred_element_type=jnp.float32)


@jax.jit
def kernel(hidden_states, v_first, x_r, x_w, x_k, x_v, x_a, x_g, k_k, k_a,
           r_k, W_r, W_k, W_v, W_o, w_lora_a, w_lora_b, w_lora_bias,
           a_lora_a, a_lora_b, a_lora_bias, v_lora_a, v_lora_b, v_lora_bias,
           g_lora_a, g_lora_b, gn_gamma, gn_beta):
    B, T, Dm = hidden_states.shape
    x = hidden_states
    shifted = jnp.pad(x, ((0, 0), (1, 0), (0, 0)))[:, :T]
    mixes = jnp.stack([x_r, x_w, x_k, x_v, x_a, x_g], axis=0)
    kka = jnp.stack([k_k, k_a], axis=0)
    biases = [w_lora_bias.reshape(1, Dm), a_lora_bias.reshape(1, Dm),
              v_lora_bias.reshape(1, Dm)]
    gn = jnp.stack([gn_gamma, gn_beta], axis=0)

    nbt = T // BS
    row_spec = pl.BlockSpec((1, BS, Dm), lambda b, t: (b, t, 0))
    full = lambda s: pl.BlockSpec(s, lambda b, t: tuple(0 for _ in s))
    outs = pl.pallas_call(
        _proj_kernel,
        grid=(B, nbt),
        in_specs=[row_spec, row_spec, row_spec,
                  full((6, Dm)), full((2, Dm)),
                  full(W_r.shape), full(W_k.shape), full(W_v.shape),
                  full(w_lora_a.shape), full(w_lora_b.shape), full((1, Dm)),
                  full(a_lora_a.shape), full(a_lora_b.shape), full((1, Dm)),
                  full(v_lora_a.shape), full(v_lora_b.shape), full((1, Dm)),
                  full(g_lora_a.shape), full(g_lora_b.shape)],
        out_specs=[row_spec] * 7,
        out_shape=[jax.ShapeDtypeStruct((B, T, Dm), jnp.float32)] * 7,
        compiler_params=pltpu.CompilerParams(
            dimension_semantics=("parallel", "parallel"),
            vmem_limit_bytes=60 * 1024 * 1024,
        ),
    )(x, shifted, v_first, mixes, kka,
      W_r.astype(_BF), W_k.astype(_BF), W_v.astype(_BF),
      w_lora_a, w_lora_b, biases[0], a_lora_a, a_lora_b, biases[1],
      v_lora_a, v_lora_b, biases[2], g_lora_a, g_lora_b)
    r, w, k, v, am, bm, g = outs

    nc = T // C
    chunk_spec = pl.BlockSpec((1, C, Dm), lambda b, t: (b, t, 0))
    out = pl.pallas_call(
        _scan_kernel,
        grid=(B, nc),
        in_specs=[chunk_spec] * 6 + [chunk_spec,
                  full(W_o.shape), full(r_k.shape), full((2, Dm))],
        out_specs=chunk_spec,
        out_shape=jax.ShapeDtypeStruct((B, T, Dm), jnp.float32),
        scratch_shapes=[pltpu.VMEM((HEADS * HK, HV), jnp.float32),
                        pltpu.VMEM((C, Dm), jnp.float32)],
        compiler_params=pltpu.CompilerParams(
            dimension_semantics=("parallel", "arbitrary"),
            vmem_limit_bytes=60 * 1024 * 1024,
        ),
    )(r, w, k, v, am, bm, g, W_o.astype(_BF), r_k, gn)
    return out


# speedup vs baseline: 6.4197x; 1.0016x over previous
import functools

import jax
import jax.numpy as jnp
from jax.experimental import pallas as pl
from jax.experimental.pallas import tpu as pltpu

D = 1024
HEADS = 16
HK = 64
HV = 64
GN_EPS = HK * 1e-5
W_SCALE = -0.6065306597126334

BS = 256
C = 64

_HI = jax.lax.Precision.HIGHEST
_BF = jnp.bfloat16


def _dot_t(x, w, precision=None):
    return jax.lax.dot_general(x, w, (((1,), (1,)), ((), ())),
                               precision=precision,
                               preferred_element_type=jnp.float32)


def _dot(x, w, precision=None):
    return jax.lax.dot_general(x, w, (((1,), (0,)), ((), ())),
                               precision=precision,
                               preferred_element_type=jnp.float32)


def _bdot_t(x, y):
    return jax.lax.dot_general(x.astype(_BF), y.astype(_BF),
                               (((1,), (1,)), ((), ())),
                               preferred_element_type=jnp.float32)


def _bdot(x, y):
    return jax.lax.dot_general(x.astype(_BF), y.astype(_BF),
                               (((1,), (0,)), ((), ())),
                               preferred_element_type=jnp.float32)


def _bdot0(x, y):
    return jax.lax.dot_general(x.astype(_BF), y.astype(_BF),
                               (((0,), (0,)), ((), ())),
                               preferred_element_type=jnp.float32)


def _proj_kernel(x_ref, xs_ref, vf_ref,
                 mixes_ref, kka_ref,
                 wr_ref, wk_ref, wv_ref,
                 wla_ref, wlb_ref, wlbias_ref,
                 ala_ref, alb_ref, albias_ref,
                 vla_ref, vlb_ref, vlbias_ref,
                 gla_ref, glb_ref,
                 r_out, w_out, k_out, v_out, am_out, bm_out, g_out):
    x = x_ref[0]
    delta = xs_ref[0] - x
    mixes = mixes_ref[...]
    xr = x + delta * mixes[0:1, :]
    xw = x + delta * mixes[1:2, :]
    xk = x + delta * mixes[2:3, :]
    xv = x + delta * mixes[3:4, :]
    xa = x + delta * mixes[4:5, :]
    xg = x + delta * mixes[5:6, :]

    r = _bdot_t(xr, wr_ref[...])
    k0 = _bdot_t(xk, wk_ref[...])
    v0 = _bdot_t(xv, wv_ref[...])

    w_log = W_SCALE * jax.nn.sigmoid(
        _dot_t(jnp.tanh(_dot_t(xw, wla_ref[...], _HI)), wlb_ref[...], _HI)
        + wlbias_ref[...])
    a_sig = jax.nn.sigmoid(
        _dot_t(_dot_t(xa, ala_ref[...], _HI), alb_ref[...], _HI)
        + albias_ref[...])
    v_mix = jax.nn.sigmoid(
        _dot_t(_dot_t(xv, vla_ref[...], _HI), vlb_ref[...], _HI)
        + vlbias_ref[...])
    g = _dot_t(jax.nn.sigmoid(_dot_t(xg, gla_ref[...], _HI)), glb_ref[...], _HI)

    v = v0 + v_mix * (vf_ref[0] - v0)

    k_k = kka_ref[0:1, :]
    k_a = kka_ref[1:2, :]
    kk = k0 * k_k
    ch = jax.lax.broadcasted_iota(jnp.int32, (D, HEADS), 0) // HK
    hh = jax.lax.broadcasted_iota(jnp.int32, (D, HEADS), 1)
    mask_dh = jnp.where(ch == hh, 1.0, 0.0)
    s2h = _dot(kk * kk, mask_dh, _HI)
    s2 = _dot_t(s2h, mask_dh, _HI)
    kk_n = kk / jnp.maximum(jnp.sqrt(s2), 1e-12)

    k = k0 * (1.0 + (a_sig - 1.0) * k_a)

    r_out[0] = r
    w_out[0] = w_log
    k_out[0] = k
    v_out[0] = v
    am_out[0] = -kk_n
    bm_out[0] = kk_n * a_sig
    g_out[0] = g


def _scan_kernel(r_ref, w_ref, k_ref, v_ref, am_ref, bm_ref, g_ref,
                 wo_ref, rk_ref, gn_ref,
                 out_ref, s_sc, y_sc):
    ci = pl.program_id(1)

    @pl.when(ci == 0)
    def _():
        s_sc[...] = jnp.zeros_like(s_sc)

    i0 = jax.lax.broadcasted_iota(jnp.int32, (C, C), 0)
    i1 = jax.lax.broadcasted_iota(jnp.int32, (C, C), 1)
    l_incl = jnp.where(i0 >= i1, 1.0, 0.0)
    eye = jnp.where(i0 == i1, 1.0, 0.0)
    i02 = jax.lax.broadcasted_iota(jnp.int32, (C, 2 * C), 0)
    j2 = jax.lax.broadcasted_iota(jnp.int32, (C, 2 * C), 1) & (C - 1)
    strict2 = i02 > j2
    incl2 = i02 >= j2

    r_all = r_ref[0]
    w_all = w_ref[0]
    k_all = k_ref[0]
    v_all = v_ref[0]
    a_all = am_ref[0]
    b_all = bm_ref[0]
    s_all = s_sc[...]
    s_news = []

    for h in range(HEADS):
        sl = slice(h * HK, (h + 1) * HK)
        rc = r_all[:, sl]
        wc = w_all[:, sl]
        kc = k_all[:, sl]
        vc = v_all[:, sl]
        ac = a_all[:, sl]
        bc = b_all[:, sl]
        S = s_all[sl, :]

        wh = wc.astype(_BF).astype(jnp.float32)
        wl = wc - wh
        c = _bdot(l_incl, wh) + _bdot(l_incl, wl)
        ec = jnp.exp(c)
        ec_inv = 1.0 / ec
        rhat = rc * ec
        ahat = ac * jnp.exp(c - wc)
        bkt = jnp.concatenate([bc * ec_inv, kc * ec_inv], axis=0)

        mm = jnp.where(strict2, _bdot_t(ahat, bkt), 0.0)
        m_ab = mm[:, :C]
        m_ak = mm[:, C:]

        tm = eye + m_ab
        mp = m_ab
        for _ in range(5):
            mp = _bdot(mp, mp)
            tm = _bdot(eye + mp, tm)

        vs = jnp.concatenate([vc, S], axis=0)
        rhs = _bdot(jnp.concatenate([m_ak, ahat], axis=1), vs)
        u = _bdot(tm, rhs)

        rr = jnp.where(incl2, _bdot_t(rhat, bkt), 0.0)
        uvs = jnp.concatenate([u, vs], axis=0)
        o = _bdot(jnp.concatenate([rr, rhat], axis=1), uvs)

        clast = c[C - 1:C, :]
        dl = jnp.exp(clast - c)
        pq = jnp.concatenate([bc * dl, kc * dl], axis=0)
        s_new = (S * jnp.transpose(jnp.exp(clast))
                 + _bdot0(pq, uvs[:2 * C]))
        s_news.append(s_new)

        mean = jnp.mean(o, axis=-1, keepdims=True)
        xc = o - mean
        var = jnp.mean(xc * xc, axis=-1, keepdims=True)
        o_n = xc * jax.lax.rsqrt(var + GN_EPS)
        gamma = gn_ref[0:1, sl]
        beta = gn_ref[1:2, sl]
        rk_row = rk_ref[h:h + 1, :]
        bonus = jnp.sum(rc * kc * rk_row, axis=-1, keepdims=True) * vc
        y_sc[:, sl] = o_n * gamma + beta + bonus

    s_sc[...] = jnp.concatenate(s_news, axis=0)
    yg = y_sc[...] * g_ref[0]
    out_ref[0] = jax.lax.dot_general(yg.astype(_BF), wo_ref[...],
                                     (((1,), (1,)), ((), ())),
                                     preferred_element_type=jnp.float32)


@jax.jit
def kernel(hidden_states, v_first, x_r, x_w, x_k, x_v, x_a, x_g, k_k, k_a,
           r_k, W_r, W_k, W_v, W_o, w_lora_a, w_lora_b, w_lora_bias,
           a_lora_a, a_lora_b, a_lora_bias, v_lora_a, v_lora_b, v_lora_bias,
           g_lora_a, g_lora_b, gn_gamma, gn_beta):
    B, T, Dm = hidden_states.shape
    x = hidden_states
    shifted = jnp.pad(x, ((0, 0), (1, 0), (0, 0)))[:, :T]
    mixes = jnp.stack([x_r, x_w, x_k, x_v, x_a, x_g], axis=0)
    kka = jnp.stack([k_k, k_a], axis=0)
    biases = [w_lora_bias.reshape(1, Dm), a_lora_bias.reshape(1, Dm),
              v_lora_bias.reshape(1, Dm)]
    gn = jnp.stack([gn_gamma, gn_beta], axis=0)

    nbt = T // BS
    row_spec = pl.BlockSpec((1, BS, Dm), lambda b, t: (b, t, 0))
    full = lambda s: pl.BlockSpec(s, lambda b, t: tuple(0 for _ in s))
    outs = pl.pallas_call(
        _proj_kernel,
        grid=(B, nbt),
        in_specs=[row_spec, row_spec, row_spec,
                  full((6, Dm)), full((2, Dm)),
                  full(W_r.shape), full(W_k.shape), full(W_v.shape),
                  full(w_lora_a.shape), full(w_lora_b.shape), full((1, Dm)),
                  full(a_lora_a.shape), full(a_lora_b.shape), full((1, Dm)),
                  full(v_lora_a.shape), full(v_lora_b.shape), full((1, Dm)),
                  full(g_lora_a.shape), full(g_lora_b.shape)],
        out_specs=[row_spec] * 7,
        out_shape=[jax.ShapeDtypeStruct((B, T, Dm), jnp.float32)] * 7,
        compiler_params=pltpu.CompilerParams(
            dimension_semantics=("parallel", "parallel"),
            vmem_limit_bytes=60 * 1024 * 1024,
        ),
    )(x, shifted, v_first, mixes, kka,
      W_r.astype(_BF), W_k.astype(_BF), W_v.astype(_BF),
      w_lora_a, w_lora_b, biases[0], a_lora_a, a_lora_b, biases[1],
      v_lora_a, v_lora_b, biases[2], g_lora_a, g_lora_b)
    r, w, k, v, am, bm, g = outs

    nc = T // C
    chunk_spec = pl.BlockSpec((1, C, Dm), lambda b, t: (b, t, 0))
    out = pl.pallas_call(
        _scan_kernel,
        grid=(B, nc),
        in_specs=[chunk_spec] * 6 + [chunk_spec,
                  full(W_o.shape), full(r_k.shape), full((2, Dm))],
        out_specs=chunk_spec,
        out_shape=jax.ShapeDtypeStruct((B, T, Dm), jnp.float32),
        scratch_shapes=[pltpu.VMEM((HEADS * HK, HV), jnp.float32),
                        pltpu.VMEM((C, Dm), jnp.float32)],
        compiler_params=pltpu.CompilerParams(
            dimension_semantics=("parallel", "arbitrary"),
            vmem_limit_bytes=60 * 1024 * 1024,
        ),
    )(r, w, k, v, am, bm, g, W_o.astype(_BF), r_k, gn)
    return out


# speedup vs baseline: 11.4116x; 1.7776x over previous
import functools

import jax
import jax.numpy as jnp
from jax.experimental import pallas as pl
from jax.experimental.pallas import tpu as pltpu

D = 1024
HEADS = 16
HK = 64
HV = 64
GN_EPS = HK * 1e-5
W_SCALE = -0.6065306597126334

BS = 256
C = 128

_HI = jax.lax.Precision.HIGHEST
_BF = jnp.bfloat16


def _dot_t(x, w, precision=None):
    return jax.lax.dot_general(x, w, (((1,), (1,)), ((), ())),
                               precision=precision,
                               preferred_element_type=jnp.float32)


def _dot(x, w, precision=None):
    return jax.lax.dot_general(x, w, (((1,), (0,)), ((), ())),
                               precision=precision,
                               preferred_element_type=jnp.float32)


def _bdot_t(x, y):
    return jax.lax.dot_general(x.astype(_BF), y.astype(_BF),
                               (((1,), (1,)), ((), ())),
                               preferred_element_type=jnp.float32)


def _bdot(x, y):
    return jax.lax.dot_general(x.astype(_BF), y.astype(_BF),
                               (((1,), (0,)), ((), ())),
                               preferred_element_type=jnp.float32)


def _bdot0(x, y):
    return jax.lax.dot_general(x.astype(_BF), y.astype(_BF),
                               (((0,), (0,)), ((), ())),
                               preferred_element_type=jnp.float32)


def _proj_kernel(x_ref, xs_ref, vf_ref,
                 mixes_ref, kka_ref,
                 wr_ref, wk_ref, wv_ref,
                 wla_ref, wlb_ref, wlbias_ref,
                 ala_ref, alb_ref, albias_ref,
                 vla_ref, vlb_ref, vlbias_ref,
                 gla_ref, glb_ref,
                 r_out, w_out, k_out, v_out, am_out, bm_out, g_out):
    x = x_ref[0]
    delta = xs_ref[0] - x
    mixes = mixes_ref[...]
    xr = x + delta * mixes[0:1, :]
    xw = x + delta * mixes[1:2, :]
    xk = x + delta * mixes[2:3, :]
    xv = x + delta * mixes[3:4, :]
    xa = x + delta * mixes[4:5, :]
    xg = x + delta * mixes[5:6, :]

    r = _bdot_t(xr, wr_ref[...])
    k0 = _bdot_t(xk, wk_ref[...])
    v0 = _bdot_t(xv, wv_ref[...])

    w_log = W_SCALE * jax.nn.sigmoid(
        _dot_t(jnp.tanh(_dot_t(xw, wla_ref[...], _HI)), wlb_ref[...], _HI)
        + wlbias_ref[...])
    a_sig = jax.nn.sigmoid(
        _dot_t(_dot_t(xa, ala_ref[...], _HI), alb_ref[...], _HI)
        + albias_ref[...])
    v_mix = jax.nn.sigmoid(
        _dot_t(_dot_t(xv, vla_ref[...], _HI), vlb_ref[...], _HI)
        + vlbias_ref[...])
    g = _dot_t(jax.nn.sigmoid(_dot_t(xg, gla_ref[...], _HI)), glb_ref[...], _HI)

    v = v0 + v_mix * (vf_ref[0] - v0)

    k_k = kka_ref[0:1, :]
    k_a = kka_ref[1:2, :]
    kk = k0 * k_k
    ch = jax.lax.broadcasted_iota(jnp.int32, (D, HEADS), 0) // HK
    hh = jax.lax.broadcasted_iota(jnp.int32, (D, HEADS), 1)
    mask_dh = jnp.where(ch == hh, 1.0, 0.0)
    s2h = _dot(kk * kk, mask_dh, _HI)
    s2 = _dot_t(s2h, mask_dh, _HI)
    kk_n = kk / jnp.maximum(jnp.sqrt(s2), 1e-12)

    k = k0 * (1.0 + (a_sig - 1.0) * k_a)

    r_out[0] = r
    w_out[0] = w_log
    k_out[0] = k
    v_out[0] = v
    am_out[0] = -kk_n
    bm_out[0] = kk_n * a_sig
    g_out[0] = g


def _scan_kernel(r_ref, w_ref, k_ref, v_ref, am_ref, bm_ref, g_ref,
                 wo_ref, rk_ref, gn_ref,
                 out_ref, s_sc, y_sc):
    ci = pl.program_id(1)

    @pl.when(ci == 0)
    def _():
        s_sc[...] = jnp.zeros_like(s_sc)

    i0 = jax.lax.broadcasted_iota(jnp.int32, (C, C), 0)
    i1 = jax.lax.broadcasted_iota(jnp.int32, (C, C), 1)
    l_incl = jnp.where(i0 >= i1, 1.0, 0.0)
    eye = jnp.where(i0 == i1, 1.0, 0.0)
    i02 = jax.lax.broadcasted_iota(jnp.int32, (2 * C, 2 * C), 0)
    j2 = jax.lax.broadcasted_iota(jnp.int32, (2 * C, 2 * C), 1) & (C - 1)
    strict2 = i02 > j2
    incl2 = (i02 - C) >= j2

    r_all = r_ref[0]
    w_all = w_ref[0]
    k_all = k_ref[0]
    v_all = v_ref[0]
    a_all = am_ref[0]
    b_all = bm_ref[0]
    s_all = s_sc[...]
    s_news = []

    w_hi = w_all.astype(_BF).astype(jnp.float32)
    w_lo = w_all - w_hi
    c_all = _bdot(l_incl, w_hi) + _bdot(l_incl, w_lo)
    ec_all = jnp.exp(c_all)
    ecinv_all = 1.0 / ec_all
    rhat_all = r_all * ec_all
    ahat_all = a_all * jnp.exp(c_all - w_all)
    btil_all = b_all * ecinv_all
    ktil_all = k_all * ecinv_all
    clast_all = c_all[C - 1:C, :]
    dl_all = jnp.exp(clast_all - c_all)

    for h in range(HEADS):
        sl = slice(h * HK, (h + 1) * HK)
        rc = r_all[:, sl]
        kc = k_all[:, sl]
        vc = v_all[:, sl]
        bc = b_all[:, sl]
        S = s_all[sl, :]
        rhat = rhat_all[:, sl]
        ahat = ahat_all[:, sl]
        clast = clast_all[:, sl]
        dl = dl_all[:, sl]
        bkt = jnp.concatenate([btil_all[:, sl], ktil_all[:, sl]], axis=0)

        ar = jnp.concatenate([ahat, rhat], axis=0)
        mr = _bdot_t(ar, bkt)
        mm = jnp.where(strict2[:C], mr[:C], 0.0)
        rr = jnp.where(incl2[C:], mr[C:], 0.0)
        m_ab = mm[:, :C]
        m_ak = mm[:, C:]

        tm = eye + m_ab
        mp = m_ab
        for _ in range(C.bit_length() - 2):
            mp = _bdot(mp, mp)
            tm = _bdot(eye + mp, tm)

        vs = jnp.concatenate([vc, S], axis=0)
        rhs = _bdot(jnp.concatenate([m_ak, ahat], axis=1), vs)
        u = _bdot(tm, rhs)

        uvs = jnp.concatenate([u, vs], axis=0)
        o = _bdot(jnp.concatenate([rr, rhat], axis=1), uvs)

        pq = jnp.concatenate([bc * dl, kc * dl], axis=0)
        s_new = (S * jnp.transpose(jnp.exp(clast))
                 + _bdot0(pq, uvs[:2 * C]))
        s_news.append(s_new)

        mean = jnp.mean(o, axis=-1, keepdims=True)
        xc = o - mean
        var = jnp.mean(xc * xc, axis=-1, keepdims=True)
        o_n = xc * jax.lax.rsqrt(var + GN_EPS)
        gamma = gn_ref[0:1, sl]
        beta = gn_ref[1:2, sl]
        rk_row = rk_ref[h:h + 1, :]
        bonus = jnp.sum(rc * kc * rk_row, axis=-1, keepdims=True) * vc
        y_sc[:, sl] = o_n * gamma + beta + bonus

    s_sc[...] = jnp.concatenate(s_news, axis=0)
    yg = y_sc[...] * g_ref[0]
    out_ref[0] = jax.lax.dot_general(yg.astype(_BF), wo_ref[...],
                                     (((1,), (1,)), ((), ())),
                                     preferred_element_type=jnp.float32)


@jax.jit
def kernel(hidden_states, v_first, x_r, x_w, x_k, x_v, x_a, x_g, k_k, k_a,
           r_k, W_r, W_k, W_v, W_o, w_lora_a, w_lora_b, w_lora_bias,
           a_lora_a, a_lora_b, a_lora_bias, v_lora_a, v_lora_b, v_lora_bias,
           g_lora_a, g_lora_b, gn_gamma, gn_beta):
    B, T, Dm = hidden_states.shape
    x = hidden_states
    shifted = jnp.pad(x, ((0, 0), (1, 0), (0, 0)))[:, :T]
    mixes = jnp.stack([x_r, x_w, x_k, x_v, x_a, x_g], axis=0)
    kka = jnp.stack([k_k, k_a], axis=0)
    biases = [w_lora_bias.reshape(1, Dm), a_lora_bias.reshape(1, Dm),
              v_lora_bias.reshape(1, Dm)]
    gn = jnp.stack([gn_gamma, gn_beta], axis=0)

    nbt = T // BS
    row_spec = pl.BlockSpec((1, BS, Dm), lambda b, t: (b, t, 0))
    full = lambda s: pl.BlockSpec(s, lambda b, t: tuple(0 for _ in s))
    outs = pl.pallas_call(
        _proj_kernel,
        grid=(B, nbt),
        in_specs=[row_spec, row_spec, row_spec,
                  full((6, Dm)), full((2, Dm)),
                  full(W_r.shape), full(W_k.shape), full(W_v.shape),
                  full(w_lora_a.shape), full(w_lora_b.shape), full((1, Dm)),
                  full(a_lora_a.shape), full(a_lora_b.shape), full((1, Dm)),
                  full(v_lora_a.shape), full(v_lora_b.shape), full((1, Dm)),
                  full(g_lora_a.shape), full(g_lora_b.shape)],
        out_specs=[row_spec] * 7,
        out_shape=[jax.ShapeDtypeStruct((B, T, Dm), jnp.float32)] * 7,
        compiler_params=pltpu.CompilerParams(
            dimension_semantics=("parallel", "parallel"),
            vmem_limit_bytes=60 * 1024 * 1024,
        ),
    )(x, shifted, v_first, mixes, kka,
      W_r.astype(_BF), W_k.astype(_BF), W_v.astype(_BF),
      w_lora_a, w_lora_b, biases[0], a_lora_a, a_lora_b, biases[1],
      v_lora_a, v_lora_b, biases[2], g_lora_a, g_lora_b)
    r, w, k, v, am, bm, g = outs

    nc = T // C
    chunk_spec = pl.BlockSpec((1, C, Dm), lambda b, t: (b, t, 0))
    out = pl.pallas_call(
        _scan_kernel,
        grid=(B, nc),
        in_specs=[chunk_spec] * 6 + [chunk_spec,
                  full(W_o.shape), full(r_k.shape), full((2, Dm))],
        out_specs=chunk_spec,
        out_shape=jax.ShapeDtypeStruct((B, T, Dm), jnp.float32),
        scratch_shapes=[pltpu.VMEM((HEADS * HK, HV), jnp.float32),
                        pltpu.VMEM((C, Dm), jnp.float32)],
        compiler_params=pltpu.CompilerParams(
            dimension_semantics=("parallel", "arbitrary"),
            vmem_limit_bytes=60 * 1024 * 1024,
        ),
    )(r, w, k, v, am, bm, g, W_o.astype(_BF), r_k, gn)
    return out


# speedup vs baseline: 12.5060x; 1.0959x over previous
import functools

import jax
import jax.numpy as jnp
from jax.experimental import pallas as pl
from jax.experimental.pallas import tpu as pltpu

D = 1024
HEADS = 16
HK = 64
HV = 64
GN_EPS = HK * 1e-5
W_SCALE = -0.6065306597126334

BS = 256
C = 128

_HI = jax.lax.Precision.HIGHEST
_BF = jnp.bfloat16


def _dot_t(x, w, precision=None):
    return jax.lax.dot_general(x, w, (((1,), (1,)), ((), ())),
                               precision=precision,
                               preferred_element_type=jnp.float32)


def _dot(x, w, precision=None):
    return jax.lax.dot_general(x, w, (((1,), (0,)), ((), ())),
                               precision=precision,
                               preferred_element_type=jnp.float32)


def _bdot_t(x, y):
    return jax.lax.dot_general(x.astype(_BF), y.astype(_BF),
                               (((1,), (1,)), ((), ())),
                               preferred_element_type=jnp.float32)


def _bdot(x, y):
    return jax.lax.dot_general(x.astype(_BF), y.astype(_BF),
                               (((1,), (0,)), ((), ())),
                               preferred_element_type=jnp.float32)


def _bdot0(x, y):
    return jax.lax.dot_general(x.astype(_BF), y.astype(_BF),
                               (((0,), (0,)), ((), ())),
                               preferred_element_type=jnp.float32)


def _split2(x):
    hi = x.astype(_BF).astype(jnp.float32)
    return hi, x - hi


def _bdot3_t(x, w):
    xh, xl = _split2(x)
    wh, wl = _split2(w)
    return _bdot_t(xh, wh) + (_bdot_t(xh, wl) + _bdot_t(xl, wh))


def _proj_kernel(x_ref, xs_ref, vf_ref,
                 mixes_ref, kka_ref,
                 wr_ref, wk_ref, wv_ref,
                 wla_ref, wlb_ref, wlbias_ref,
                 ala_ref, alb_ref, albias_ref,
                 vla_ref, vlb_ref, vlbias_ref,
                 gla_ref, glb_ref,
                 r_out, w_out, k_out, v_out, am_out, bm_out, g_out):
    x = x_ref[0]
    delta = xs_ref[0] - x
    mixes = mixes_ref[...]
    xr = x + delta * mixes[0:1, :]
    xw = x + delta * mixes[1:2, :]
    xk = x + delta * mixes[2:3, :]
    xv = x + delta * mixes[3:4, :]
    xa = x + delta * mixes[4:5, :]
    xg = x + delta * mixes[5:6, :]

    r = _bdot_t(xr, wr_ref[...])
    k0 = _bdot_t(xk, wk_ref[...])
    v0 = _bdot_t(xv, wv_ref[...])

    w_log = W_SCALE * jax.nn.sigmoid(
        _bdot3_t(jnp.tanh(_bdot3_t(xw, wla_ref[...])), wlb_ref[...])
        + wlbias_ref[...])
    a_sig = jax.nn.sigmoid(
        _bdot3_t(_bdot3_t(xa, ala_ref[...]), alb_ref[...])
        + albias_ref[...])
    v_mix = jax.nn.sigmoid(
        _bdot3_t(_bdot3_t(xv, vla_ref[...]), vlb_ref[...])
        + vlbias_ref[...])
    g = _bdot3_t(jax.nn.sigmoid(_bdot3_t(xg, gla_ref[...])), glb_ref[...])

    v = v0 + v_mix * (vf_ref[0] - v0)

    k_k = kka_ref[0:1, :]
    k_a = kka_ref[1:2, :]
    kk = k0 * k_k
    ch = jax.lax.broadcasted_iota(jnp.int32, (D, HEADS), 0) // HK
    hh = jax.lax.broadcasted_iota(jnp.int32, (D, HEADS), 1)
    mask_dh = jnp.where(ch == hh, 1.0, 0.0)
    kk2h, kk2l = _split2(kk * kk)
    s2h = _bdot(kk2h, mask_dh) + _bdot(kk2l, mask_dh)
    s2hh, s2hl = _split2(s2h)
    s2 = _bdot_t(s2hh, mask_dh) + _bdot_t(s2hl, mask_dh)
    kk_n = kk / jnp.maximum(jnp.sqrt(s2), 1e-12)

    k = k0 * (1.0 + (a_sig - 1.0) * k_a)

    r_out[0] = r.astype(_BF)
    w_out[0] = w_log
    k_out[0] = k.astype(_BF)
    v_out[0] = v.astype(_BF)
    am_out[0] = (-kk_n).astype(_BF)
    bm_out[0] = (kk_n * a_sig).astype(_BF)
    g_out[0] = g.astype(_BF)


def _scan_kernel(r_ref, w_ref, k_ref, v_ref, am_ref, bm_ref, g_ref,
                 wo_ref, rk_ref, gn_ref,
                 out_ref, s_sc, y_sc):
    ci = pl.program_id(1)

    @pl.when(ci == 0)
    def _():
        s_sc[...] = jnp.zeros_like(s_sc)

    i0 = jax.lax.broadcasted_iota(jnp.int32, (C, C), 0)
    i1 = jax.lax.broadcasted_iota(jnp.int32, (C, C), 1)
    l_incl = jnp.where(i0 >= i1, 1.0, 0.0)
    eye = jnp.where(i0 == i1, 1.0, 0.0)
    i02 = jax.lax.broadcasted_iota(jnp.int32, (2 * C, 2 * C), 0)
    j2 = jax.lax.broadcasted_iota(jnp.int32, (2 * C, 2 * C), 1) & (C - 1)
    strict2 = i02 > j2
    incl2 = (i02 - C) >= j2

    r_all = r_ref[0]
    w_all = w_ref[0]
    k_all = k_ref[0]
    v_all = v_ref[0]
    a_all = am_ref[0]
    b_all = bm_ref[0]
    s_all = s_sc[...]
    s_news = []

    w_hi = w_all.astype(_BF).astype(jnp.float32)
    w_lo = w_all - w_hi
    c_all = _bdot(l_incl, w_hi) + _bdot(l_incl, w_lo)
    ec_all = jnp.exp(c_all)
    ecinv_all = 1.0 / ec_all
    rhat_all = r_all * ec_all
    ahat_all = a_all * jnp.exp(c_all - w_all)
    btil_all = b_all * ecinv_all
    ktil_all = k_all * ecinv_all
    clast_all = c_all[C - 1:C, :]
    dl_all = jnp.exp(clast_all - c_all)

    for h in range(HEADS):
        sl = slice(h * HK, (h + 1) * HK)
        rc = r_all[:, sl]
        kc = k_all[:, sl]
        vc = v_all[:, sl]
        bc = b_all[:, sl]
        S = s_all[sl, :]
        rhat = rhat_all[:, sl]
        ahat = ahat_all[:, sl]
        clast = clast_all[:, sl]
        dl = dl_all[:, sl]
        bkt = jnp.concatenate([btil_all[:, sl], ktil_all[:, sl]], axis=0)

        ar = jnp.concatenate([ahat, rhat], axis=0)
        mr = _bdot_t(ar, bkt)
        mm = jnp.where(strict2[:C], mr[:C], 0.0)
        rr = jnp.where(incl2[C:], mr[C:], 0.0)
        m_ab = mm[:, :C]
        m_ak = mm[:, C:]

        tm = eye + m_ab
        mp = m_ab
        for _ in range(C.bit_length() - 2):
            mp = _bdot(mp, mp)
            tm = _bdot(eye + mp, tm)

        vs = jnp.concatenate([vc, S.astype(_BF)], axis=0)
        rhs = _bdot(jnp.concatenate([m_ak, ahat], axis=1), vs)
        u = _bdot(tm, rhs)

        uvs = jnp.concatenate([u.astype(_BF), vs], axis=0)
        o = _bdot(jnp.concatenate([rr, rhat], axis=1), uvs)

        pq = jnp.concatenate([bc * dl, kc * dl], axis=0)
        s_new = (S * jnp.transpose(jnp.exp(clast))
                 + _bdot0(pq, uvs[:2 * C]))
        s_news.append(s_new)

        mean = jnp.mean(o, axis=-1, keepdims=True)
        xc = o - mean
        var = jnp.mean(xc * xc, axis=-1, keepdims=True)
        o_n = xc * jax.lax.rsqrt(var + GN_EPS)
        gamma = gn_ref[0:1, sl]
        beta = gn_ref[1:2, sl]
        rk_row = rk_ref[h:h + 1, :]
        bonus = jnp.sum(rc * kc * rk_row, axis=-1, keepdims=True) * vc
        y_sc[:, sl] = o_n * gamma + beta + bonus

    s_sc[...] = jnp.concatenate(s_news, axis=0)
    yg = y_sc[...] * g_ref[0]
    out_ref[0] = jax.lax.dot_general(yg.astype(_BF), wo_ref[...],
                                     (((1,), (1,)), ((), ())),
                                     preferred_element_type=jnp.float32)


@jax.jit
def kernel(hidden_states, v_first, x_r, x_w, x_k, x_v, x_a, x_g, k_k, k_a,
           r_k, W_r, W_k, W_v, W_o, w_lora_a, w_lora_b, w_lora_bias,
           a_lora_a, a_lora_b, a_lora_bias, v_lora_a, v_lora_b, v_lora_bias,
           g_lora_a, g_lora_b, gn_gamma, gn_beta):
    B, T, Dm = hidden_states.shape
    x = hidden_states
    shifted = jnp.pad(x, ((0, 0), (1, 0), (0, 0)))[:, :T]
    mixes = jnp.stack([x_r, x_w, x_k, x_v, x_a, x_g], axis=0)
    kka = jnp.stack([k_k, k_a], axis=0)
    biases = [w_lora_bias.reshape(1, Dm), a_lora_bias.reshape(1, Dm),
              v_lora_bias.reshape(1, Dm)]
    gn = jnp.stack([gn_gamma, gn_beta], axis=0)

    nbt = T // BS
    row_spec = pl.BlockSpec((1, BS, Dm), lambda b, t: (b, t, 0))
    full = lambda s: pl.BlockSpec(s, lambda b, t: tuple(0 for _ in s))
    outs = pl.pallas_call(
        _proj_kernel,
        grid=(B, nbt),
        in_specs=[row_spec, row_spec, row_spec,
                  full((6, Dm)), full((2, Dm)),
                  full(W_r.shape), full(W_k.shape), full(W_v.shape),
                  full(w_lora_a.shape), full(w_lora_b.shape), full((1, Dm)),
                  full(a_lora_a.shape), full(a_lora_b.shape), full((1, Dm)),
                  full(v_lora_a.shape), full(v_lora_b.shape), full((1, Dm)),
                  full(g_lora_a.shape), full(g_lora_b.shape)],
        out_specs=[row_spec] * 7,
        out_shape=[jax.ShapeDtypeStruct((B, T, Dm), dt) for dt in
                   (_BF, jnp.float32, _BF, _BF, _BF, _BF, _BF)],
        compiler_params=pltpu.CompilerParams(
            dimension_semantics=("parallel", "parallel"),
            vmem_limit_bytes=60 * 1024 * 1024,
        ),
    )(x, shifted, v_first, mixes, kka,
      W_r.astype(_BF), W_k.astype(_BF), W_v.astype(_BF),
      w_lora_a, w_lora_b, biases[0], a_lora_a, a_lora_b, biases[1],
      v_lora_a, v_lora_b, biases[2], g_lora_a, g_lora_b)
    r, w, k, v, am, bm, g = outs

    nc = T // C
    chunk_spec = pl.BlockSpec((1, C, Dm), lambda b, t: (b, t, 0))
    out = pl.pallas_call(
        _scan_kernel,
        grid=(B, nc),
        in_specs=[chunk_spec] * 6 + [chunk_spec,
                  full(W_o.shape), full(r_k.shape), full((2, Dm))],
        out_specs=chunk_spec,
        out_shape=jax.ShapeDtypeStruct((B, T, Dm), jnp.float32),
        scratch_shapes=[pltpu.VMEM((HEADS * HK, HV), jnp.float32),
                        pltpu.VMEM((C, Dm), jnp.float32)],
        compiler_params=pltpu.CompilerParams(
            dimension_semantics=("parallel", "arbitrary"),
            vmem_limit_bytes=60 * 1024 * 1024,
        ),
    )(r, w, k, v, am, bm, g, W_o.astype(_BF), r_k, gn)
    return out


# speedup vs baseline: 15.0182x; 1.2009x over previous
import functools

import jax
import jax.numpy as jnp
from jax.experimental import pallas as pl
from jax.experimental.pallas import tpu as pltpu

D = 1024
HEADS = 16
HK = 64
HV = 64
GN_EPS = HK * 1e-5
W_SCALE = -0.6065306597126334

BS = 256
C = 128

_HI = jax.lax.Precision.HIGHEST
_BF = jnp.bfloat16


def _dot_t(x, w, precision=None):
    return jax.lax.dot_general(x, w, (((1,), (1,)), ((), ())),
                               precision=precision,
                               preferred_element_type=jnp.float32)


def _dot(x, w, precision=None):
    return jax.lax.dot_general(x, w, (((1,), (0,)), ((), ())),
                               precision=precision,
                               preferred_element_type=jnp.float32)


def _bdot_t(x, y):
    return jax.lax.dot_general(x.astype(_BF), y.astype(_BF),
                               (((1,), (1,)), ((), ())),
                               preferred_element_type=jnp.float32)


def _bdot(x, y):
    return jax.lax.dot_general(x.astype(_BF), y.astype(_BF),
                               (((1,), (0,)), ((), ())),
                               preferred_element_type=jnp.float32)


def _bdot0(x, y):
    return jax.lax.dot_general(x.astype(_BF), y.astype(_BF),
                               (((0,), (0,)), ((), ())),
                               preferred_element_type=jnp.float32)


def _split2(x):
    hi = x.astype(_BF).astype(jnp.float32)
    return hi, x - hi


def _bdot3_t(x, w):
    xh, xl = _split2(x)
    wh, wl = _split2(w)
    return _bdot_t(xh, wh) + (_bdot_t(xh, wl) + _bdot_t(xl, wh))


def _proj_kernel(x_ref, xs_ref, vf_ref,
                 mixes_ref, kka_ref,
                 wr_ref, wk_ref, wv_ref,
                 wla_ref, wlb_ref, wlbias_ref,
                 ala_ref, alb_ref, albias_ref,
                 vla_ref, vlb_ref, vlbias_ref,
                 gla_ref, glb_ref,
                 r_out, w_out, k_out, v_out, am_out, bm_out, g_out):
    x = x_ref[0]
    delta = xs_ref[0] - x
    mixes = mixes_ref[...]
    xr = x + delta * mixes[0:1, :]
    xw = x + delta * mixes[1:2, :]
    xk = x + delta * mixes[2:3, :]
    xv = x + delta * mixes[3:4, :]
    xa = x + delta * mixes[4:5, :]
    xg = x + delta * mixes[5:6, :]

    r = _bdot_t(xr, wr_ref[...])
    k0 = _bdot_t(xk, wk_ref[...])
    v0 = _bdot_t(xv, wv_ref[...])

    w_log = W_SCALE * jax.nn.sigmoid(
        _bdot3_t(jnp.tanh(_bdot3_t(xw, wla_ref[...])), wlb_ref[...])
        + wlbias_ref[...])
    a_sig = jax.nn.sigmoid(
        _bdot3_t(_bdot3_t(xa, ala_ref[...]), alb_ref[...])
        + albias_ref[...])
    v_mix = jax.nn.sigmoid(
        _bdot3_t(_bdot3_t(xv, vla_ref[...]), vlb_ref[...])
        + vlbias_ref[...])
    g = _bdot3_t(jax.nn.sigmoid(_bdot3_t(xg, gla_ref[...])), glb_ref[...])

    v = v0 + v_mix * (vf_ref[0] - v0)

    k_k = kka_ref[0:1, :]
    k_a = kka_ref[1:2, :]
    kk = k0 * k_k
    ch = jax.lax.broadcasted_iota(jnp.int32, (D, HEADS), 0) // HK
    hh = jax.lax.broadcasted_iota(jnp.int32, (D, HEADS), 1)
    mask_dh = jnp.where(ch == hh, 1.0, 0.0)
    kk2h, kk2l = _split2(kk * kk)
    s2h = _bdot(kk2h, mask_dh) + _bdot(kk2l, mask_dh)
    s2hh, s2hl = _split2(s2h)
    s2 = _bdot_t(s2hh, mask_dh) + _bdot_t(s2hl, mask_dh)
    kk_n = kk / jnp.maximum(jnp.sqrt(s2), 1e-12)

    k = k0 * (1.0 + (a_sig - 1.0) * k_a)

    r_out[0] = r.astype(_BF)
    w_out[0] = w_log
    k_out[0] = k.astype(_BF)
    v_out[0] = v.astype(_BF)
    am_out[0] = (-kk_n).astype(_BF)
    bm_out[0] = (kk_n * a_sig).astype(_BF)
    g_out[0] = g.astype(_BF)


def _scan_kernel(r_ref, w_ref, k_ref, v_ref, am_ref, bm_ref, g_ref,
                 wo_ref, rk_ref, gn_ref,
                 out_ref, s_sc, y_sc):
    ci = pl.program_id(1)

    @pl.when(ci == 0)
    def _():
        s_sc[...] = jnp.zeros_like(s_sc)

    i0 = jax.lax.broadcasted_iota(jnp.int32, (C, C), 0)
    i1 = jax.lax.broadcasted_iota(jnp.int32, (C, C), 1)
    l_incl = jnp.where(i0 >= i1, 1.0, 0.0)
    eye = jnp.where(i0 == i1, 1.0, 0.0)
    i02 = jax.lax.broadcasted_iota(jnp.int32, (2 * C, 2 * C), 0)
    j2 = jax.lax.broadcasted_iota(jnp.int32, (2 * C, 2 * C), 1) & (C - 1)
    strict2 = i02 > j2
    incl2 = (i02 - C) >= j2

    r_all = r_ref[0]
    w_all = w_ref[0]
    k_all = k_ref[0]
    v_all = v_ref[0]
    a_all = am_ref[0]
    b_all = bm_ref[0]
    s_all = s_sc[...]
    s_news = []

    w_hi = w_all.astype(_BF).astype(jnp.float32)
    w_lo = w_all - w_hi
    c_all = _bdot(l_incl, w_hi) + _bdot(l_incl, w_lo)
    ec_all = jnp.exp(c_all)
    ecinv_all = 1.0 / ec_all
    rhat_all = r_all * ec_all
    ahat_all = a_all * jnp.exp(c_all - w_all)
    btil_all = b_all * ecinv_all
    ktil_all = k_all * ecinv_all
    clast_all = c_all[C - 1:C, :]
    dl_all = jnp.exp(clast_all - c_all)

    zero_c = jnp.zeros((C, C), jnp.float32)
    eye2 = jnp.where(i02 == jax.lax.broadcasted_iota(
        jnp.int32, (2 * C, 2 * C), 1), 1.0, 0.0)

    for p in range(HEADS // 2):
        pre = []
        for h in (2 * p, 2 * p + 1):
            sl = slice(h * HK, (h + 1) * HK)
            S = s_all[sl, :]
            rhat = rhat_all[:, sl]
            ahat = ahat_all[:, sl]
            bkt = jnp.concatenate([btil_all[:, sl], ktil_all[:, sl]], axis=0)

            ar = jnp.concatenate([ahat, rhat], axis=0)
            mr = _bdot_t(ar, bkt)
            mm = jnp.where(strict2[:C], mr[:C], 0.0)
            rr = jnp.where(incl2[C:], mr[C:], 0.0)

            vs = jnp.concatenate([v_all[:, sl], S.astype(_BF)], axis=0)
            rhs = _bdot(jnp.concatenate([mm[:, C:], ahat], axis=1), vs)
            pre.append((sl, S, rhat, mm[:, :C], rr, vs, rhs))

        bd = jnp.concatenate(
            [jnp.concatenate([pre[0][3], zero_c], axis=1),
             jnp.concatenate([zero_c, pre[1][3]], axis=1)], axis=0)
        tm = eye2 + bd
        mp = bd
        for _ in range(C.bit_length() - 2):
            mp = _bdot(mp, mp)
            tm = _bdot(eye2 + mp, tm)
        u2 = _bdot(tm, jnp.concatenate([pre[0][6], pre[1][6]], axis=0))

        for i, (sl, S, rhat, _, rr, vs, _) in enumerate(pre):
            u = u2[i * C:(i + 1) * C]
            rc = r_all[:, sl]
            kc = k_all[:, sl]
            vc = v_all[:, sl]
            bc = b_all[:, sl]
            clast = clast_all[:, sl]
            dl = dl_all[:, sl]

            uvs = jnp.concatenate([u.astype(_BF), vs], axis=0)
            o = _bdot(jnp.concatenate([rr, rhat], axis=1), uvs)

            pq = jnp.concatenate([bc * dl, kc * dl], axis=0)
            s_new = (S * jnp.transpose(jnp.exp(clast))
                     + _bdot0(pq, uvs[:2 * C]))
            s_news.append(s_new)

            mean = jnp.mean(o, axis=-1, keepdims=True)
            xc = o - mean
            var = jnp.mean(xc * xc, axis=-1, keepdims=True)
            o_n = xc * jax.lax.rsqrt(var + GN_EPS)
            h = 2 * p + i
            gamma = gn_ref[0:1, sl]
            beta = gn_ref[1:2, sl]
            rk_row = rk_ref[h:h + 1, :]
            bonus = jnp.sum(rc * kc * rk_row, axis=-1, keepdims=True) * vc
            y_sc[:, sl] = o_n * gamma + beta + bonus

    s_sc[...] = jnp.concatenate(s_news, axis=0)
    yg = y_sc[...] * g_ref[0]
    out_ref[0] = jax.lax.dot_general(yg.astype(_BF), wo_ref[...],
                                     (((1,), (1,)), ((), ())),
                                     preferred_element_type=jnp.float32)


@jax.jit
def kernel(hidden_states, v_first, x_r, x_w, x_k, x_v, x_a, x_g, k_k, k_a,
           r_k, W_r, W_k, W_v, W_o, w_lora_a, w_lora_b, w_lora_bias,
           a_lora_a, a_lora_b, a_lora_bias, v_lora_a, v_lora_b, v_lora_bias,
           g_lora_a, g_lora_b, gn_gamma, gn_beta):
    B, T, Dm = hidden_states.shape
    x = hidden_states
    shifted = jnp.pad(x, ((0, 0), (1, 0), (0, 0)))[:, :T]
    mixes = jnp.stack([x_r, x_w, x_k, x_v, x_a, x_g], axis=0)
    kka = jnp.stack([k_k, k_a], axis=0)
    biases = [w_lora_bias.reshape(1, Dm), a_lora_bias.reshape(1, Dm),
              v_lora_bias.reshape(1, Dm)]
    gn = jnp.stack([gn_gamma, gn_beta], axis=0)

    nbt = T // BS
    row_spec = pl.BlockSpec((1, BS, Dm), lambda b, t: (b, t, 0))
    full = lambda s: pl.BlockSpec(s, lambda b, t: tuple(0 for _ in s))
    outs = pl.pallas_call(
        _proj_kernel,
        grid=(B, nbt),
        in_specs=[row_spec, row_spec, row_spec,
                  full((6, Dm)), full((2, Dm)),
                  full(W_r.shape), full(W_k.shape), full(W_v.shape),
                  full(w_lora_a.shape), full(w_lora_b.shape), full((1, Dm)),
                  full(a_lora_a.shape), full(a_lora_b.shape), full((1, Dm)),
                  full(v_lora_a.shape), full(v_lora_b.shape), full((1, Dm)),
                  full(g_lora_a.shape), full(g_lora_b.shape)],
        out_specs=[row_spec] * 7,
        out_shape=[jax.ShapeDtypeStruct((B, T, Dm), dt) for dt in
                   (_BF, jnp.float32, _BF, _BF, _BF, _BF, _BF)],
        compiler_params=pltpu.CompilerParams(
            dimension_semantics=("parallel", "parallel"),
            vmem_limit_bytes=60 * 1024 * 1024,
        ),
    )(x, shifted, v_first, mixes, kka,
      W_r.astype(_BF), W_k.astype(_BF), W_v.astype(_BF),
      w_lora_a, w_lora_b, biases[0], a_lora_a, a_lora_b, biases[1],
      v_lora_a, v_lora_b, biases[2], g_lora_a, g_lora_b)
    r, w, k, v, am, bm, g = outs

    nc = T // C
    chunk_spec = pl.BlockSpec((1, C, Dm), lambda b, t: (b, t, 0))
    out = pl.pallas_call(
        _scan_kernel,
        grid=(B, nc),
        in_specs=[chunk_spec] * 6 + [chunk_spec,
                  full(W_o.shape), full(r_k.shape), full((2, Dm))],
        out_specs=chunk_spec,
        out_shape=jax.ShapeDtypeStruct((B, T, Dm), jnp.float32),
        scratch_shapes=[pltpu.VMEM((HEADS * HK, HV), jnp.float32),
                        pltpu.VMEM((C, Dm), jnp.float32)],
        compiler_params=pltpu.CompilerParams(
            dimension_semantics=("parallel", "arbitrary"),
            vmem_limit_bytes=60 * 1024 * 1024,
        ),
    )(r, w, k, v, am, bm, g, W_o.astype(_BF), r_k, gn)
    return out


# speedup vs baseline: 15.2024x; 1.0123x over previous
import functools

import jax
import jax.numpy as jnp
from jax.experimental import pallas as pl
from jax.experimental.pallas import tpu as pltpu

D = 1024
HEADS = 16
HK = 64
HV = 64
GN_EPS = HK * 1e-5
W_SCALE = -0.6065306597126334

BS = 512
C = 128

_HI = jax.lax.Precision.HIGHEST
_BF = jnp.bfloat16


def _dot_t(x, w, precision=None):
    return jax.lax.dot_general(x, w, (((1,), (1,)), ((), ())),
                               precision=precision,
                               preferred_element_type=jnp.float32)


def _dot(x, w, precision=None):
    return jax.lax.dot_general(x, w, (((1,), (0,)), ((), ())),
                               precision=precision,
                               preferred_element_type=jnp.float32)


def _bdot_t(x, y):
    return jax.lax.dot_general(x.astype(_BF), y.astype(_BF),
                               (((1,), (1,)), ((), ())),
                               preferred_element_type=jnp.float32)


def _bdot(x, y):
    return jax.lax.dot_general(x.astype(_BF), y.astype(_BF),
                               (((1,), (0,)), ((), ())),
                               preferred_element_type=jnp.float32)


def _bdot0(x, y):
    return jax.lax.dot_general(x.astype(_BF), y.astype(_BF),
                               (((0,), (0,)), ((), ())),
                               preferred_element_type=jnp.float32)


def _split2(x):
    hi = x.astype(_BF).astype(jnp.float32)
    return hi, x - hi


def _bdot3_t(x, w):
    xh, xl = _split2(x)
    wh, wl = _split2(w)
    return _bdot_t(xh, wh) + (_bdot_t(xh, wl) + _bdot_t(xl, wh))


def _proj_kernel(x_ref, xs_ref, vf_ref,
                 mixes_ref, kka_ref,
                 wr_ref, wk_ref, wv_ref,
                 wla_ref, wlb_ref, wlbias_ref,
                 ala_ref, alb_ref, albias_ref,
                 vla_ref, vlb_ref, vlbias_ref,
                 gla_ref, glb_ref,
                 r_out, w_out, k_out, v_out, am_out, bm_out, g_out):
    x = x_ref[0]
    delta = xs_ref[0] - x
    mixes = mixes_ref[...]
    xr = x + delta * mixes[0:1, :]
    xw = x + delta * mixes[1:2, :]
    xk = x + delta * mixes[2:3, :]
    xv = x + delta * mixes[3:4, :]
    xa = x + delta * mixes[4:5, :]
    xg = x + delta * mixes[5:6, :]

    r = _bdot_t(xr, wr_ref[...])
    k0 = _bdot_t(xk, wk_ref[...])
    v0 = _bdot_t(xv, wv_ref[...])

    w_log = W_SCALE * jax.nn.sigmoid(
        _bdot3_t(jnp.tanh(_bdot3_t(xw, wla_ref[...])), wlb_ref[...])
        + wlbias_ref[...])
    a_sig = jax.nn.sigmoid(
        _bdot3_t(_bdot3_t(xa, ala_ref[...]), alb_ref[...])
        + albias_ref[...])
    v_mix = jax.nn.sigmoid(
        _bdot3_t(_bdot3_t(xv, vla_ref[...]), vlb_ref[...])
        + vlbias_ref[...])
    g = _bdot3_t(jax.nn.sigmoid(_bdot3_t(xg, gla_ref[...])), glb_ref[...])

    v = v0 + v_mix * (vf_ref[0] - v0)

    k_k = kka_ref[0:1, :]
    k_a = kka_ref[1:2, :]
    kk = k0 * k_k
    ch = jax.lax.broadcasted_iota(jnp.int32, (D, HEADS), 0) // HK
    hh = jax.lax.broadcasted_iota(jnp.int32, (D, HEADS), 1)
    mask_dh = jnp.where(ch == hh, 1.0, 0.0)
    kk2h, kk2l = _split2(kk * kk)
    s2h = _bdot(kk2h, mask_dh) + _bdot(kk2l, mask_dh)
    s2hh, s2hl = _split2(s2h)
    s2 = _bdot_t(s2hh, mask_dh) + _bdot_t(s2hl, mask_dh)
    kk_n = kk / jnp.maximum(jnp.sqrt(s2), 1e-12)

    k = k0 * (1.0 + (a_sig - 1.0) * k_a)

    r_out[0] = r.astype(_BF)
    w_out[0] = w_log
    k_out[0] = k.astype(_BF)
    v_out[0] = v.astype(_BF)
    am_out[0] = (-kk_n).astype(_BF)
    bm_out[0] = (kk_n * a_sig).astype(_BF)
    g_out[0] = g.astype(_BF)


def _scan_kernel(r_ref, w_ref, k_ref, v_ref, am_ref, bm_ref, g_ref,
                 wo_ref, rk_ref, gn_ref,
                 out_ref, s_sc, y_sc):
    ci = pl.program_id(1)

    @pl.when(ci == 0)
    def _():
        s_sc[...] = jnp.zeros_like(s_sc)

    i0 = jax.lax.broadcasted_iota(jnp.int32, (C, C), 0)
    i1 = jax.lax.broadcasted_iota(jnp.int32, (C, C), 1)
    l_incl = jnp.where(i0 >= i1, 1.0, 0.0)
    eye = jnp.where(i0 == i1, 1.0, 0.0)
    i02 = jax.lax.broadcasted_iota(jnp.int32, (2 * C, 2 * C), 0)
    j2 = jax.lax.broadcasted_iota(jnp.int32, (2 * C, 2 * C), 1) & (C - 1)
    strict2 = i02 > j2
    incl2 = (i02 - C) >= j2

    r_all = r_ref[0]
    w_all = w_ref[0]
    k_all = k_ref[0]
    v_all = v_ref[0]
    a_all = am_ref[0]
    b_all = bm_ref[0]
    s_all = s_sc[...]
    s_news = []

    w_hi = w_all.astype(_BF).astype(jnp.float32)
    w_lo = w_all - w_hi
    c_all = _bdot(l_incl, w_hi) + _bdot(l_incl, w_lo)
    ec_all = jnp.exp(c_all)
    ecinv_all = 1.0 / ec_all
    rhat_all = r_all * ec_all
    ahat_all = a_all * jnp.exp(c_all - w_all)
    btil_all = b_all * ecinv_all
    ktil_all = k_all * ecinv_all
    clast_all = c_all[C - 1:C, :]
    dl_all = jnp.exp(clast_all - c_all)

    zero_c = jnp.zeros((C, C), jnp.float32)
    eye2 = jnp.where(i02 == jax.lax.broadcasted_iota(
        jnp.int32, (2 * C, 2 * C), 1), 1.0, 0.0)

    for p in range(HEADS // 2):
        pre = []
        for h in (2 * p, 2 * p + 1):
            sl = slice(h * HK, (h + 1) * HK)
            S = s_all[sl, :]
            rhat = rhat_all[:, sl]
            ahat = ahat_all[:, sl]
            bkt = jnp.concatenate([btil_all[:, sl], ktil_all[:, sl]], axis=0)

            ar = jnp.concatenate([ahat, rhat], axis=0)
            mr = _bdot_t(ar, bkt)
            mm = jnp.where(strict2[:C], mr[:C], 0.0)
            rr = jnp.where(incl2[C:], mr[C:], 0.0)

            vs = jnp.concatenate([v_all[:, sl], S.astype(_BF)], axis=0)
            rhs = _bdot(jnp.concatenate([mm[:, C:], ahat], axis=1), vs)
            pre.append((sl, S, rhat, mm[:, :C], rr, vs, rhs))

        bd = jnp.concatenate(
            [jnp.concatenate([pre[0][3], zero_c], axis=1),
             jnp.concatenate([zero_c, pre[1][3]], axis=1)], axis=0)
        tm = eye2 + bd
        mp = bd
        for _ in range(C.bit_length() - 2):
            mp = _bdot(mp, mp)
            tm = _bdot(eye2 + mp, tm)
        u2 = _bdot(tm, jnp.concatenate([pre[0][6], pre[1][6]], axis=0))

        for i, (sl, S, rhat, _, rr, vs, _) in enumerate(pre):
            u = u2[i * C:(i + 1) * C]
            rc = r_all[:, sl]
            kc = k_all[:, sl]
            vc = v_all[:, sl]
            bc = b_all[:, sl]
            clast = clast_all[:, sl]
            dl = dl_all[:, sl]

            uvs = jnp.concatenate([u.astype(_BF), vs], axis=0)
            o = _bdot(jnp.concatenate([rr, rhat], axis=1), uvs)

            pq = jnp.concatenate([bc * dl, kc * dl], axis=0)
            s_new = (S * jnp.transpose(jnp.exp(clast))
                     + _bdot0(pq, uvs[:2 * C]))
            s_news.append(s_new)

            mean = jnp.mean(o, axis=-1, keepdims=True)
            xc = o - mean
            var = jnp.mean(xc * xc, axis=-1, keepdims=True)
            o_n = xc * jax.lax.rsqrt(var + GN_EPS)
            h = 2 * p + i
            gamma = gn_ref[0:1, sl]
            beta = gn_ref[1:2, sl]
            rk_row = rk_ref[h:h + 1, :]
            bonus = jnp.sum(rc * kc * rk_row, axis=-1, keepdims=True) * vc
            y_sc[:, sl] = o_n * gamma + beta + bonus

    s_sc[...] = jnp.concatenate(s_news, axis=0)
    yg = y_sc[...] * g_ref[0]
    out_ref[0] = jax.lax.dot_general(yg.astype(_BF), wo_ref[...],
                                     (((1,), (1,)), ((), ())),
                                     preferred_element_type=jnp.float32)


@jax.jit
def kernel(hidden_states, v_first, x_r, x_w, x_k, x_v, x_a, x_g, k_k, k_a,
           r_k, W_r, W_k, W_v, W_o, w_lora_a, w_lora_b, w_lora_bias,
           a_lora_a, a_lora_b, a_lora_bias, v_lora_a, v_lora_b, v_lora_bias,
           g_lora_a, g_lora_b, gn_gamma, gn_beta):
    B, T, Dm = hidden_states.shape
    x = hidden_states
    shifted = jnp.pad(x, ((0, 0), (1, 0), (0, 0)))[:, :T]
    mixes = jnp.stack([x_r, x_w, x_k, x_v, x_a, x_g], axis=0)
    kka = jnp.stack([k_k, k_a], axis=0)
    biases = [w_lora_bias.reshape(1, Dm), a_lora_bias.reshape(1, Dm),
              v_lora_bias.reshape(1, Dm)]
    gn = jnp.stack([gn_gamma, gn_beta], axis=0)

    nbt = T // BS
    row_spec = pl.BlockSpec((1, BS, Dm), lambda b, t: (b, t, 0))
    full = lambda s: pl.BlockSpec(s, lambda b, t: tuple(0 for _ in s))
    outs = pl.pallas_call(
        _proj_kernel,
        grid=(B, nbt),
        in_specs=[row_spec, row_spec, row_spec,
                  full((6, Dm)), full((2, Dm)),
                  full(W_r.shape), full(W_k.shape), full(W_v.shape),
                  full(w_lora_a.shape), full(w_lora_b.shape), full((1, Dm)),
                  full(a_lora_a.shape), full(a_lora_b.shape), full((1, Dm)),
                  full(v_lora_a.shape), full(v_lora_b.shape), full((1, Dm)),
                  full(g_lora_a.shape), full(g_lora_b.shape)],
        out_specs=[row_spec] * 7,
        out_shape=[jax.ShapeDtypeStruct((B, T, Dm), dt) for dt in
                   (_BF, jnp.float32, _BF, _BF, _BF, _BF, _BF)],
        compiler_params=pltpu.CompilerParams(
            dimension_semantics=("parallel", "parallel"),
            vmem_limit_bytes=60 * 1024 * 1024,
        ),
    )(x, shifted, v_first, mixes, kka,
      W_r.astype(_BF), W_k.astype(_BF), W_v.astype(_BF),
      w_lora_a, w_lora_b, biases[0], a_lora_a, a_lora_b, biases[1],
      v_lora_a, v_lora_b, biases[2], g_lora_a, g_lora_b)
    r, w, k, v, am, bm, g = outs

    nc = T // C
    chunk_spec = pl.BlockSpec((1, C, Dm), lambda b, t: (b, t, 0))
    out = pl.pallas_call(
        _scan_kernel,
        grid=(B, nc),
        in_specs=[chunk_spec] * 6 + [chunk_spec,
                  full(W_o.shape), full(r_k.shape), full((2, Dm))],
        out_specs=chunk_spec,
        out_shape=jax.ShapeDtypeStruct((B, T, Dm), jnp.float32),
        scratch_shapes=[pltpu.VMEM((HEADS * HK, HV), jnp.float32),
                        pltpu.VMEM((C, Dm), jnp.float32)],
        compiler_params=pltpu.CompilerParams(
            dimension_semantics=("parallel", "arbitrary"),
            vmem_limit_bytes=60 * 1024 * 1024,
        ),
    )(r, w, k, v, am, bm, g, W_o.astype(_BF), r_k, gn)
    return out
